# Initial kernel scaffold; baseline (speedup 1.0000x reference)
import jax
import jax.numpy as jnp
from jax.experimental import pallas as pl


def kernel(x, edge_index, edge_attr, W1, b1, g1, be1, W2, b2, g2, be2, cW1, cb1, cg1, cbe1, cW2, cb2, cg2, cbe2, cW3, cb3, cW4, cb4):
    raise NotImplementedError("write your pallas kernel here")


# speedup vs baseline: 3.9404x; 3.9404x over previous
import functools

import jax
import jax.numpy as jnp
from jax import lax
from jax.experimental import pallas as pl
from jax.experimental.pallas import tpu as pltpu
from jax.experimental.pallas import tpu_sc as plsc

F32 = jnp.float32
NC = 2
NS = 16
CH = 128


def _mesh():
    return plsc.VectorSubcoreMesh(
        core_axis_name="c", subcore_axis_name="s", num_cores=NC, num_subcores=NS
    )


def _sc_degree(dst_pad, n_pad):
    e_pad = dst_pad.shape[0]
    ew = e_pad // (NC * NS)
    nchunk = ew // CH
    rps = n_pad // NS

    @functools.partial(
        pl.kernel,
        out_type=(
            jax.ShapeDtypeStruct((n_pad, 16), F32),
            jax.ShapeDtypeStruct((n_pad, 16), F32),
        ),
        mesh=_mesh(),
        scratch_types=dict(
            ones_v=pltpu.VMEM((CH, 16), F32),
            idx_v=pltpu.VMEM((CH,), jnp.int32),
            obuf=pltpu.VMEM((rps, 16), F32),
            acc=pltpu.VMEM_SHARED((n_pad, 16), F32),
        ),
    )
    def k(dst_h, deg_a, deg_b, ones_v, idx_v, obuf, acc):
        c = lax.axis_index("c")
        s = lax.axis_index("s")
        wid = s * NC + c

        def fill_ones(i, carry):
            ones_v[i, :] = jnp.ones((16,), F32)
            return carry

        lax.fori_loop(0, CH, fill_ones, 0)

        def fill_zero(i, carry):
            obuf[i, :] = jnp.zeros((16,), F32)
            return carry

        lax.fori_loop(0, rps, fill_zero, 0)
        pltpu.sync_copy(obuf, acc.at[pl.ds(s * rps, rps)])
        plsc.subcore_barrier()

        base0 = wid * ew

        def chunk(i, carry):
            b = base0 + i * CH
            pltpu.sync_copy(dst_h.at[pl.ds(b, CH)], idx_v)
            pltpu.sync_copy(ones_v, acc.at[idx_v], add=True)
            return carry

        lax.fori_loop(0, nchunk, chunk, 0)
        plsc.subcore_barrier()

        pltpu.sync_copy(acc.at[pl.ds(s * rps, rps)], obuf)

        @pl.when(c == 0)
        def _():
            pltpu.sync_copy(obuf, deg_a.at[pl.ds(s * rps, rps)])

        @pl.when(c == 1)
        def _():
            pltpu.sync_copy(obuf, deg_b.at[pl.ds(s * rps, rps)])

    return k(dst_pad)


def _sc_aggregate_es(y, src_pad, dst_pad):
    n_pad, w = y.shape
    e_pad = src_pad.shape[0]
    ew = e_pad // (NC * NS)
    nchunk = ew // CH
    rps = n_pad // NS

    @functools.partial(
        pl.kernel,
        out_type=(
            jax.ShapeDtypeStruct((n_pad, w), F32),
            jax.ShapeDtypeStruct((n_pad, w), F32),
        ),
        mesh=_mesh(),
        scratch_types=dict(
            rows_v=pltpu.VMEM((CH, w), F32),
            idx_s=pltpu.VMEM((CH,), jnp.int32),
            idx_d=pltpu.VMEM((CH,), jnp.int32),
            acc=pltpu.VMEM_SHARED((n_pad, w), F32),
            sem=pltpu.SemaphoreType.DMA,
        ),
    )
    def k(y_h, src_h, dst_h, agg_a, agg_b, rows_v, idx_s, idx_d, acc, sem):
        c = lax.axis_index("c")
        s = lax.axis_index("s")
        wid = s * NC + c

        def fill_zero(i, carry):
            for j in range(w // 16):
                rows_v[i, pl.ds(j * 16, 16)] = jnp.zeros((16,), F32)
            return carry

        lax.fori_loop(0, CH, fill_zero, 0)
        for r in range(rps // CH):
            pltpu.sync_copy(rows_v, acc.at[pl.ds(s * rps + r * CH, CH)])
        plsc.subcore_barrier()

        base0 = wid * ew

        def chunk(i, carry):
            b = base0 + i * CH
            pltpu.sync_copy(src_h.at[pl.ds(b, CH)], idx_s)
            pltpu.sync_copy(dst_h.at[pl.ds(b, CH)], idx_d)
            pltpu.async_copy(y_h.at[idx_s], rows_v, sem).wait()
            pltpu.sync_copy(rows_v, acc.at[idx_d], add=True)
            return carry

        lax.fori_loop(0, nchunk, chunk, 0)
        plsc.subcore_barrier()

        def out_piece(i, carry):
            off = s * rps + i * CH
            pltpu.sync_copy(acc.at[pl.ds(off, CH)], rows_v)

            @pl.when(c == 0)
            def _():
                pltpu.sync_copy(rows_v, agg_a.at[pl.ds(off, CH)])

            @pl.when(c == 1)
            def _():
                pltpu.sync_copy(rows_v, agg_b.at[pl.ds(off, CH)])

            return carry

        lax.fori_loop(0, rps // CH, out_piece, 0)

    return k(y, src_pad, dst_pad)


def _sc_aggregate_cs(y_a, y_b, src_pad, dst_pad):
    n_pad, w = y_a.shape
    e_pad = src_pad.shape[0]
    ew = e_pad // NS
    nchunk = ew // CH
    rps = n_pad // NS

    @functools.partial(
        pl.kernel,
        out_type=(
            jax.ShapeDtypeStruct((n_pad, w), F32),
            jax.ShapeDtypeStruct((n_pad, w), F32),
        ),
        mesh=_mesh(),
        scratch_types=dict(
            rows_v=pltpu.VMEM((CH, w), F32),
            idx_s=pltpu.VMEM((CH,), jnp.int32),
            idx_d=pltpu.VMEM((CH,), jnp.int32),
            acc=pltpu.VMEM_SHARED((n_pad, w), F32),
            sem=pltpu.SemaphoreType.DMA,
        ),
    )
    def k(ya_h, yb_h, src_h, dst_h, agg_a, agg_b, rows_v, idx_s, idx_d, acc, sem):
        c = lax.axis_index("c")
        s = lax.axis_index("s")

        def fill_zero(i, carry):
            for j in range(w // 16):
                rows_v[i, pl.ds(j * 16, 16)] = jnp.zeros((16,), F32)
            return carry

        lax.fori_loop(0, CH, fill_zero, 0)
        for r in range(rps // CH):
            pltpu.sync_copy(rows_v, acc.at[pl.ds(s * rps + r * CH, CH)])
        plsc.subcore_barrier()

        base0 = s * ew

        def body(y_h):
            def chunk(i, carry):
                b = base0 + i * CH
                pltpu.sync_copy(src_h.at[pl.ds(b, CH)], idx_s)
                pltpu.sync_copy(dst_h.at[pl.ds(b, CH)], idx_d)
                pltpu.async_copy(y_h.at[idx_s], rows_v, sem).wait()
                pltpu.sync_copy(rows_v, acc.at[idx_d], add=True)
                return carry

            lax.fori_loop(0, nchunk, chunk, 0)

        @pl.when(c == 0)
        def _():
            body(ya_h)

        @pl.when(c == 1)
        def _():
            body(yb_h)

        plsc.subcore_barrier()

        def out_piece(i, carry):
            off = s * rps + i * CH
            pltpu.sync_copy(acc.at[pl.ds(off, CH)], rows_v)

            @pl.when(c == 0)
            def _():
                pltpu.sync_copy(rows_v, agg_a.at[pl.ds(off, CH)])

            @pl.when(c == 1)
            def _():
                pltpu.sync_copy(rows_v, agg_b.at[pl.ds(off, CH)])

            return carry

        lax.fori_loop(0, rps // CH, out_piece, 0)

    return k(y_a, y_b, src_pad, dst_pad)


def _sc_edge_gather(p, q, src_pad, dst_pad):
    n_pad, w = p.shape
    e_pad = src_pad.shape[0]
    ew = e_pad // (NC * NS)
    nchunk = ew // CH

    @functools.partial(
        pl.kernel,
        out_type=(
            jax.ShapeDtypeStruct((e_pad, w), F32),
            jax.ShapeDtypeStruct((e_pad, w), F32),
        ),
        mesh=_mesh(),
        scratch_types=dict(
            buf_p=pltpu.VMEM((CH, w), F32),
            buf_q=pltpu.VMEM((CH, w), F32),
            idx_s=pltpu.VMEM((CH,), jnp.int32),
            idx_d=pltpu.VMEM((CH,), jnp.int32),
            sem=pltpu.SemaphoreType.DMA,
        ),
    )
    def k(p_h, q_h, src_h, dst_h, tp, tq, buf_p, buf_q, idx_s, idx_d, sem):
        c = lax.axis_index("c")
        s = lax.axis_index("s")
        wid = s * NC + c
        base0 = wid * ew

        def chunk(i, carry):
            b = base0 + i * CH
            pltpu.sync_copy(src_h.at[pl.ds(b, CH)], idx_s)
            pltpu.sync_copy(dst_h.at[pl.ds(b, CH)], idx_d)
            pltpu.async_copy(p_h.at[idx_s], buf_p, sem).wait()
            pltpu.async_copy(q_h.at[idx_d], buf_q, sem).wait()
            pltpu.sync_copy(buf_p, tp.at[pl.ds(b, CH)])
            pltpu.sync_copy(buf_q, tq.at[pl.ds(b, CH)])
            return carry

        lax.fori_loop(0, nchunk, chunk, 0)

    return k(p, q, src_pad, dst_pad)


_RB = 256


def _tc_conv1(x_pad, w1, deg_a, deg_b):
    n_pad, f = x_pad.shape
    h = w1.shape[0]

    def body(x_r, w_r, da_r, db_r, y_r, dv_r):
        xw = lax.dot_general(
            x_r[...], w_r[...], (((1,), (1,)), ((), ())),
            preferred_element_type=F32,
        )
        deg = da_r[:, :1] + db_r[:, :1] + 1.0
        dinv = lax.rsqrt(deg)
        y_r[...] = xw * dinv
        dv_r[...] = jnp.broadcast_to(dinv, (_RB, 8))

    grid = (n_pad // _RB,)
    return pl.pallas_call(
        body,
        grid=grid,
        in_specs=[
            pl.BlockSpec((_RB, f), lambda i: (i, 0)),
            pl.BlockSpec((h, f), lambda i: (0, 0)),
            pl.BlockSpec((_RB, 16), lambda i: (i, 0)),
            pl.BlockSpec((_RB, 16), lambda i: (i, 0)),
        ],
        out_specs=[
            pl.BlockSpec((_RB, h), lambda i: (i, 0)),
            pl.BlockSpec((_RB, 8), lambda i: (i, 0)),
        ],
        out_shape=[
            jax.ShapeDtypeStruct((n_pad, h), F32),
            jax.ShapeDtypeStruct((n_pad, 8), F32),
        ],
    )(x_pad, w1, deg_a, deg_b)


def _tc_conv2(agg_a, agg_b, y1, dinv8, w2t, s1, c1):
    n_pad, hw = agg_a.shape
    h2w = w2t.shape[1]

    def body(aa_r, ab_r, y_r, dv_r, w_r, s_r, c_r, oa_r, ob_r):
        dinv = dv_r[:, :1]
        pre = (aa_r[...] + ab_r[...] + y_r[...]) * dinv
        h1 = jnp.maximum(pre * s_r[...] + c_r[...], 0.0)
        xw2 = lax.dot_general(
            h1, w_r[...], (((1,), (0,)), ((), ())), preferred_element_type=F32
        )
        y2 = xw2 * dinv
        oa_r[...] = y2[:, : h2w // 2]
        ob_r[...] = y2[:, h2w // 2:]

    grid = (n_pad // _RB,)
    return pl.pallas_call(
        body,
        grid=grid,
        in_specs=[
            pl.BlockSpec((_RB, hw), lambda i: (i, 0)),
            pl.BlockSpec((_RB, hw), lambda i: (i, 0)),
            pl.BlockSpec((_RB, hw), lambda i: (i, 0)),
            pl.BlockSpec((_RB, 8), lambda i: (i, 0)),
            pl.BlockSpec(w2t.shape, lambda i: (0, 0)),
            pl.BlockSpec((1, hw), lambda i: (0, 0)),
            pl.BlockSpec((1, hw), lambda i: (0, 0)),
        ],
        out_specs=[
            pl.BlockSpec((_RB, h2w // 2), lambda i: (i, 0)),
            pl.BlockSpec((_RB, h2w // 2), lambda i: (i, 0)),
        ],
        out_shape=[
            jax.ShapeDtypeStruct((n_pad, h2w // 2), F32),
            jax.ShapeDtypeStruct((n_pad, h2w // 2), F32),
        ],
    )(agg_a, agg_b, y1, dinv8, w2t, s1, c1)


def _tc_node_proj(agg_a, agg_b, y_a, y_b, dinv8, m_src, m_dst, s2, c2):
    n_pad, hw = agg_a.shape
    ow = m_src.shape[1]

    def body(aa_r, ab_r, ya_r, yb_r, dv_r, ms_r, md_r, s_r, c_r, p_r, q_r):
        dinv = dv_r[:, :1]
        left = (aa_r[...] + ya_r[...]) * dinv
        right = (ab_r[...] + yb_r[...]) * dinv
        pre = jnp.concatenate([left, right], axis=1)
        h2 = jnp.maximum(pre * s_r[...] + c_r[...], 0.0)
        p_r[...] = lax.dot_general(
            h2, ms_r[...], (((1,), (0,)), ((), ())), preferred_element_type=F32
        )
        q_r[...] = lax.dot_general(
            h2, md_r[...], (((1,), (0,)), ((), ())), preferred_element_type=F32
        )

    grid = (n_pad // _RB,)
    return pl.pallas_call(
        body,
        grid=grid,
        in_specs=[
            pl.BlockSpec((_RB, hw), lambda i: (i, 0)),
            pl.BlockSpec((_RB, hw), lambda i: (i, 0)),
            pl.BlockSpec((_RB, hw), lambda i: (i, 0)),
            pl.BlockSpec((_RB, hw), lambda i: (i, 0)),
            pl.BlockSpec((_RB, 8), lambda i: (i, 0)),
            pl.BlockSpec(m_src.shape, lambda i: (0, 0)),
            pl.BlockSpec(m_dst.shape, lambda i: (0, 0)),
            pl.BlockSpec((1, 2 * hw), lambda i: (0, 0)),
            pl.BlockSpec((1, 2 * hw), lambda i: (0, 0)),
        ],
        out_specs=[
            pl.BlockSpec((_RB, ow), lambda i: (i, 0)),
            pl.BlockSpec((_RB, ow), lambda i: (i, 0)),
        ],
        out_shape=[
            jax.ShapeDtypeStruct((n_pad, ow), F32),
            jax.ShapeDtypeStruct((n_pad, ow), F32),
        ],
    )(agg_a, agg_b, y_a, y_b, dinv8, m_src, m_dst, s2, c2)


_EB = 512


def _tc_edge_mlp(tp, tq, ea8, m_e8, b1f, wt2, b2f, wt3, b3, wt4, b4):
    e_pad, w = tp.shape

    def body(tp_r, tq_r, ea_r, me_r, b1_r, w2_r, b2_r, w3_r, b3_r, w4_r, b4_r,
             o_r):
        et = lax.dot_general(
            ea_r[...], me_r[...], (((1,), (0,)), ((), ())),
            preferred_element_type=F32,
        )
        z1 = jnp.maximum(tp_r[...] + tq_r[...] + et + b1_r[...], 0.0)
        z2 = jnp.maximum(
            lax.dot_general(z1, w2_r[...], (((1,), (0,)), ((), ())),
                            preferred_element_type=F32) + b2_r[...], 0.0)
        z3 = jnp.maximum(
            lax.dot_general(z2, w3_r[...], (((1,), (0,)), ((), ())),
                            preferred_element_type=F32) + b3_r[...], 0.0)
        z4 = lax.dot_general(z3, w4_r[...], (((1,), (0,)), ((), ())),
                             preferred_element_type=F32) + b4_r[...]
        o_r[...] = jax.nn.sigmoid(z4)

    grid = (e_pad // _EB,)
    return pl.pallas_call(
        body,
        grid=grid,
        in_specs=[
            pl.BlockSpec((_EB, w), lambda i: (i, 0)),
            pl.BlockSpec((_EB, w), lambda i: (i, 0)),
            pl.BlockSpec((_EB, 8), lambda i: (i, 0)),
            pl.BlockSpec(m_e8.shape, lambda i: (0, 0)),
            pl.BlockSpec(b1f.shape, lambda i: (0, 0)),
            pl.BlockSpec(wt2.shape, lambda i: (0, 0)),
            pl.BlockSpec(b2f.shape, lambda i: (0, 0)),
            pl.BlockSpec(wt3.shape, lambda i: (0, 0)),
            pl.BlockSpec(b3.shape, lambda i: (0, 0)),
            pl.BlockSpec(wt4.shape, lambda i: (0, 0)),
            pl.BlockSpec(b4.shape, lambda i: (0, 0)),
        ],
        out_specs=pl.BlockSpec((_EB, 8), lambda i: (i, 0)),
        out_shape=jax.ShapeDtypeStruct((e_pad, 8), F32),
    )(tp, tq, ea8, m_e8, b1f, wt2, b2f, wt3, b3, wt4, b4)


def kernel(x, edge_index, edge_attr, W1, b1, g1, be1, W2, b2, g2, be2,
           cW1, cb1, cg1, cbe1, cW2, cb2, cg2, cbe2, cW3, cb3, cW4, cb4):
    n, f_in = x.shape
    e = edge_index.shape[1]
    h = W1.shape[0]
    h2 = W2.shape[0]

    n_pad = ((n + _RB - 1) // _RB) * _RB
    step = NC * NS * CH
    e_pad = ((e + step - 1) // step) * step
    if e_pad % _EB:
        e_pad = ((e_pad + _EB - 1) // _EB) * _EB

    src = edge_index[0]
    dst = edge_index[1]
    pad_idx = jnp.full((e_pad - e,), n, dtype=src.dtype)
    src_p = jnp.concatenate([src, pad_idx])
    dst_p = jnp.concatenate([dst, pad_idx])
    x_p = jnp.pad(x, ((0, n_pad - n), (0, 0)))
    ea8 = jnp.pad(edge_attr[:, :2], ((0, e_pad - e), (0, 6)))

    bnf = lax.rsqrt(jnp.asarray(1.0 + 1e-5, F32))
    s1 = (g1 * bnf).reshape(1, h)
    c1 = (s1[0] * b1 + be1).reshape(1, h)
    s2 = (g2 * bnf).reshape(1, h2)
    c2 = (s2[0] * b2 + be2).reshape(1, h2)

    sc1 = cg1 * bnf
    w1f = sc1[:, None] * cW1
    b1f = (sc1 * cb1 + cbe1).reshape(1, -1)
    m_src = w1f[:, :h2].T
    m_dst = w1f[:, h2:2 * h2].T
    m_e8 = jnp.pad(w1f[:, 2 * h2:].T, ((0, 6), (0, 0)))

    sc2 = cg2 * bnf
    wt2 = (sc2[:, None] * cW2).T
    b2f = (sc2 * cb2 + cbe2).reshape(1, -1)
    wt3 = cW3.T
    b3 = cb3.reshape(1, -1)
    wt4 = jnp.pad(cW4.T, ((0, 0), (0, 7)))
    b4 = jnp.broadcast_to(cb4.reshape(1, 1), (1, 8))

    w2t = W2.T

    deg_a, deg_b = _sc_degree(dst_p, n_pad)
    y1, dinv8 = _tc_conv1(x_p, W1, deg_a, deg_b)
    agg1a, agg1b = _sc_aggregate_es(y1, src_p, dst_p)
    y2a, y2b = _tc_conv2(agg1a, agg1b, y1, dinv8, w2t, s1, c1)
    agg2a, agg2b = _sc_aggregate_cs(y2a, y2b, src_p, dst_p)
    p, q = _tc_node_proj(agg2a, agg2b, y2a, y2b, dinv8, m_src, m_dst, s2, c2)
    tp, tq = _sc_edge_gather(p, q, src_p, dst_p)
    out8 = _tc_edge_mlp(tp, tq, ea8, m_e8, b1f, wt2, b2f, wt3, b3, wt4, b4)
    return out8[:e, :1]
